# baseline (speedup 1.0000x reference)
import functools

import jax
import jax.numpy as jnp
from jax import lax
from jax.experimental import pallas as pl
from jax.experimental.pallas import tpu as pltpu
from jax.experimental.pallas import tpu_sc as plsc

N_FIELDS = 26
VOCAB = 100000
EMB = 50
EMBP = 64
BATCH = 16384
LOG2_BATCH = 14
TOTAL = N_FIELDS * BATCH
NW = 32
PER_W = TOTAL // NW
CHUNK = 1024
N_CHUNKS = PER_W // CHUNK
LANES = 16
IDXW = 128
IDX_ROWS = CHUNK // IDXW


def _sc_body(x_hbm, tab_hbm, out_hbm, idx_v, rows_v, sem):
  wid = lax.axis_index("s") * 2 + lax.axis_index("c")
  base = wid * PER_W
  lane = lax.iota(jnp.int32, LANES)

  def chunk_body(k, _):
    cbase = base + k * CHUNK
    pltpu.sync_copy(x_hbm.at[pl.ds(cbase // IDXW, IDX_ROWS)], idx_v)

    def adjust(r, _):
      for c in range(IDXW // LANES):
        g = cbase + r * IDXW + c * LANES + lane
        f = lax.shift_right_logical(g, LOG2_BATCH)
        idx_v[r, pl.ds(c * LANES, LANES)] = (
            idx_v[r, pl.ds(c * LANES, LANES)] + f * VOCAB
        )
      return 0

    lax.fori_loop(0, IDX_ROWS, adjust, 0)

    copies = [
        pltpu.async_copy(
            tab_hbm.at[idx_v.at[j]],
            rows_v.at[pl.ds(j * IDXW, IDXW)],
            sem,
        )
        for j in range(IDX_ROWS)
    ]
    for cp in copies:
      cp.wait()

    def relu_row(r, _):
      for c in range(0, EMBP, LANES):
        rows_v[r, pl.ds(c, LANES)] = jnp.maximum(
            rows_v[r, pl.ds(c, LANES)], 0.0
        )
      return 0

    lax.fori_loop(0, CHUNK, relu_row, 0)

    pltpu.sync_copy(rows_v, out_hbm.at[pl.ds(cbase, CHUNK)])
    return 0

  lax.fori_loop(0, N_CHUNKS, chunk_body, 0)


@jax.jit
def kernel(x, tables):
  x_flat = x.reshape(TOTAL // IDXW, IDXW).astype(jnp.int32)
  tab_pad = jnp.pad(tables, ((0, 0), (0, 0), (0, EMBP - EMB))).reshape(
      N_FIELDS * VOCAB, EMBP
  )
  mesh = plsc.VectorSubcoreMesh(core_axis_name="c", subcore_axis_name="s")
  out = pl.kernel(
      _sc_body,
      out_type=jax.ShapeDtypeStruct((TOTAL, EMBP), jnp.float32),
      mesh=mesh,
      scratch_types=[
          pltpu.VMEM((IDX_ROWS, IDXW), jnp.int32),
          pltpu.VMEM((CHUNK, EMBP), jnp.float32),
          pltpu.SemaphoreType.DMA,
      ],
      compiler_params=pltpu.CompilerParams(use_tc_tiling_on_sc=False),
  )(x_flat, tab_pad)
  return out[:, :EMB].reshape(N_FIELDS, BATCH, EMB)


# speedup vs baseline: 4.8690x; 4.8690x over previous
import functools

import jax
import jax.numpy as jnp
from jax import lax
from jax.experimental import pallas as pl
from jax.experimental.pallas import tpu as pltpu
from jax.experimental.pallas import tpu_sc as plsc

N_FIELDS = 26
VOCAB = 100000
EMB = 50
BATCH = 16384
NW = 32
PAIRS = N_FIELDS * EMB
PAIRS_PER_W = -(-PAIRS // NW)
LANES = 16
BCHUNK = 4096
N_BCHUNK = BATCH // BCHUNK


def _sc_body(x_hbm, tab_hbm, out_hbm, row_v, idx_v, out_v):
  wid = lax.axis_index("s") * 2 + lax.axis_index("c")

  def pair_body(i, _):
    pair = wid + i * NW
    f = pair // EMB
    d = pair % EMB

    @pl.when(pair < PAIRS)
    def _():
      pltpu.sync_copy(tab_hbm.at[f, d], row_v)

      def bchunk_body(cb, _):
        pltpu.sync_copy(x_hbm.at[f, pl.ds(cb * BCHUNK, BCHUNK)], idx_v)

        def gather16(j, _):
          v16 = idx_v[pl.ds(j * LANES, LANES)]
          vals = plsc.load_gather(row_v, [v16])
          out_v[pl.ds(j * LANES, LANES)] = jnp.maximum(vals, 0.0)
          return 0

        lax.fori_loop(0, BCHUNK // LANES, gather16, 0)
        pltpu.sync_copy(out_v, out_hbm.at[f, d, pl.ds(cb * BCHUNK, BCHUNK)])
        return 0

      lax.fori_loop(0, N_BCHUNK, bchunk_body, 0)

    return 0

  lax.fori_loop(0, PAIRS_PER_W, pair_body, 0)


@jax.jit
def kernel(x, tables):
  xi = x.astype(jnp.int32)
  tab_t = jnp.transpose(tables, (0, 2, 1))
  mesh = plsc.VectorSubcoreMesh(core_axis_name="c", subcore_axis_name="s")
  out_t = pl.kernel(
      _sc_body,
      out_type=jax.ShapeDtypeStruct((N_FIELDS, EMB, BATCH), jnp.float32),
      mesh=mesh,
      scratch_types=[
          pltpu.VMEM((VOCAB,), jnp.float32),
          pltpu.VMEM((BCHUNK,), jnp.int32),
          pltpu.VMEM((BCHUNK,), jnp.float32),
      ],
      compiler_params=pltpu.CompilerParams(
          use_tc_tiling_on_sc=True, needs_layout_passes=False
      ),
  )(xi, tab_t)
  return jnp.transpose(out_t, (0, 2, 1))


# speedup vs baseline: 6.7381x; 1.3839x over previous
import functools

import jax
import jax.numpy as jnp
from jax import lax
from jax.experimental import pallas as pl
from jax.experimental.pallas import tpu as pltpu
from jax.experimental.pallas import tpu_sc as plsc

N_FIELDS = 26
VOCAB = 100000
EMB = 50
BATCH = 16384
NW = 32
PAIRS = N_FIELDS * EMB
PAIRS_PER_W = -(-PAIRS // NW)
LANES = 16
BCHUNK = 4096
N_BCHUNK = BATCH // BCHUNK


def _sc_body(x_hbm, tab_hbm, out_hbm, row_v, idx_v, out_v):
  wid = lax.axis_index("s") * 2 + lax.axis_index("c")

  def pair_body(i, _):
    pair = wid + i * NW
    f = pair // EMB
    d = pair % EMB

    @pl.when(pair < PAIRS)
    def _():
      pltpu.sync_copy(tab_hbm.at[f, d], row_v)

      def bchunk_body(cb, _):
        pltpu.sync_copy(x_hbm.at[f, pl.ds(cb * BCHUNK, BCHUNK)], idx_v)

        @plsc.parallel_loop(0, BCHUNK, step=LANES * 4, unroll=4)
        def _(j):
          for u in range(4):
            v16 = idx_v[pl.ds(j + u * LANES, LANES)]
            vals = plsc.load_gather(row_v, [v16])
            out_v[pl.ds(j + u * LANES, LANES)] = jnp.maximum(vals, 0.0)

        pltpu.sync_copy(out_v, out_hbm.at[f, d, pl.ds(cb * BCHUNK, BCHUNK)])
        return 0

      lax.fori_loop(0, N_BCHUNK, bchunk_body, 0)

    return 0

  lax.fori_loop(0, PAIRS_PER_W, pair_body, 0)


@jax.jit
def kernel(x, tables):
  xi = x.astype(jnp.int32)
  tab_t = jnp.transpose(tables, (0, 2, 1))
  mesh = plsc.VectorSubcoreMesh(core_axis_name="c", subcore_axis_name="s")
  out_t = pl.kernel(
      _sc_body,
      out_type=jax.ShapeDtypeStruct((N_FIELDS, EMB, BATCH), jnp.float32),
      mesh=mesh,
      scratch_types=[
          pltpu.VMEM((VOCAB,), jnp.float32),
          pltpu.VMEM((BCHUNK,), jnp.int32),
          pltpu.VMEM((BCHUNK,), jnp.float32),
      ],
      compiler_params=pltpu.CompilerParams(
          use_tc_tiling_on_sc=True, needs_layout_passes=False
      ),
  )(xi, tab_t)
  return jnp.transpose(out_t, (0, 2, 1))


# speedup vs baseline: 7.8075x; 1.1587x over previous
import functools

import jax
import jax.numpy as jnp
from jax import lax
from jax.experimental import pallas as pl
from jax.experimental.pallas import tpu as pltpu
from jax.experimental.pallas import tpu_sc as plsc

N_FIELDS = 26
VOCAB = 100000
EMB = 50
BATCH = 16384
NW = 32
PAIRS = N_FIELDS * EMB
PAIRS_PER_W = -(-PAIRS // NW)
LANES = 16
BCHUNK = 4096
N_BCHUNK = BATCH // BCHUNK


def _sc_body(x_hbm, tab_hbm, out_hbm, row_v, idx_v, out_v, isem, osem):
  wid = lax.axis_index("s") * 2 + lax.axis_index("c")

  def pair_body(i, _):
    pair = wid + i * NW
    f = pair // EMB
    d = pair % EMB

    @pl.when(pair < PAIRS)
    def _():
      pltpu.sync_copy(tab_hbm.at[f, d], row_v)
      pltpu.async_copy(
          x_hbm.at[f, pl.ds(0, BCHUNK)], idx_v.at[0], isem.at[0]
      )
      for cb in range(N_BCHUNK):
        p = cb % 2
        pltpu.make_async_copy(
            x_hbm.at[f, pl.ds(cb * BCHUNK, BCHUNK)], idx_v.at[p], isem.at[p]
        ).wait()
        if cb + 1 < N_BCHUNK:
          pltpu.async_copy(
              x_hbm.at[f, pl.ds((cb + 1) * BCHUNK, BCHUNK)],
              idx_v.at[1 - p],
              isem.at[1 - p],
          )
        if cb >= 2:
          pltpu.make_async_copy(
              out_v.at[p],
              out_hbm.at[f, d, pl.ds((cb - 2) * BCHUNK, BCHUNK)],
              osem.at[p],
          ).wait()

        @plsc.parallel_loop(0, BCHUNK, step=LANES * 4, unroll=4)
        def _(j):
          for u in range(4):
            v16 = idx_v[p, pl.ds(j + u * LANES, LANES)]
            vals = plsc.load_gather(row_v, [v16])
            out_v[p, pl.ds(j + u * LANES, LANES)] = jnp.maximum(vals, 0.0)

        pltpu.async_copy(
            out_v.at[p], out_hbm.at[f, d, pl.ds(cb * BCHUNK, BCHUNK)], osem.at[p]
        )
      for cb in (N_BCHUNK - 2, N_BCHUNK - 1):
        pltpu.make_async_copy(
            out_v.at[cb % 2],
            out_hbm.at[f, d, pl.ds(cb * BCHUNK, BCHUNK)],
            osem.at[cb % 2],
        ).wait()

    return 0

  lax.fori_loop(0, PAIRS_PER_W, pair_body, 0)


@jax.jit
def kernel(x, tables):
  xi = x.astype(jnp.int32)
  tab_t = jnp.transpose(tables, (0, 2, 1))
  mesh = plsc.VectorSubcoreMesh(core_axis_name="c", subcore_axis_name="s")
  out_t = pl.kernel(
      _sc_body,
      out_type=jax.ShapeDtypeStruct((N_FIELDS, EMB, BATCH), jnp.float32),
      mesh=mesh,
      scratch_types=[
          pltpu.VMEM((VOCAB,), jnp.float32),
          pltpu.VMEM((2, BCHUNK), jnp.int32),
          pltpu.VMEM((2, BCHUNK), jnp.float32),
          pltpu.SemaphoreType.DMA((2,)),
          pltpu.SemaphoreType.DMA((2,)),
      ],
      compiler_params=pltpu.CompilerParams(
          use_tc_tiling_on_sc=True, needs_layout_passes=False
      ),
  )(xi, tab_t)
  return jnp.transpose(out_t, (0, 2, 1))


# speedup vs baseline: 8.4195x; 1.0784x over previous
import functools

import jax
import jax.numpy as jnp
from jax import lax
from jax.experimental import pallas as pl
from jax.experimental.pallas import tpu as pltpu
from jax.experimental.pallas import tpu_sc as plsc

N_FIELDS = 26
VOCAB = 100000
EMB = 50
BATCH = 16384
NW = 32
PAIRS = N_FIELDS * EMB
PAIRS_PER_W = -(-PAIRS // NW)
LANES = 16
BCHUNK = 4096
N_BCHUNK = BATCH // BCHUNK


def _sc_body(x_hbm, tab_hbm, out_hbm, row_v, idx_v, out_v, isem, osem, rsem):
  wid = lax.axis_index("s") * 2 + lax.axis_index("c")

  def pair_body(i, _):
    pair = wid + i * NW
    f = pair // EMB
    d = pair % EMB

    @pl.when(pair < PAIRS)
    def _():
      pltpu.async_copy(tab_hbm.at[f, d], row_v, rsem.at[0])
      pltpu.async_copy(
          x_hbm.at[f, pl.ds(0, BCHUNK)], idx_v.at[0], isem.at[0]
      )
      pltpu.make_async_copy(tab_hbm.at[f, d], row_v, rsem.at[0]).wait()
      for cb in range(N_BCHUNK):
        p = cb % 2
        pltpu.make_async_copy(
            x_hbm.at[f, pl.ds(cb * BCHUNK, BCHUNK)], idx_v.at[p], isem.at[p]
        ).wait()
        if cb + 1 < N_BCHUNK:
          pltpu.async_copy(
              x_hbm.at[f, pl.ds((cb + 1) * BCHUNK, BCHUNK)],
              idx_v.at[1 - p],
              isem.at[1 - p],
          )
        if cb >= 2:
          pltpu.make_async_copy(
              out_v.at[p],
              out_hbm.at[f, d, pl.ds((cb - 2) * BCHUNK, BCHUNK)],
              osem.at[p],
          ).wait()

        @plsc.parallel_loop(0, BCHUNK, step=LANES * 4, unroll=4)
        def _(j):
          for u in range(4):
            v16 = idx_v[p, pl.ds(j + u * LANES, LANES)]
            vals = plsc.load_gather(row_v, [v16])
            out_v[p, pl.ds(j + u * LANES, LANES)] = jnp.maximum(vals, 0.0)

        pltpu.async_copy(
            out_v.at[p], out_hbm.at[f, d, pl.ds(cb * BCHUNK, BCHUNK)], osem.at[p]
        )
      for cb in (N_BCHUNK - 2, N_BCHUNK - 1):
        pltpu.make_async_copy(
            out_v.at[cb % 2],
            out_hbm.at[f, d, pl.ds(cb * BCHUNK, BCHUNK)],
            osem.at[cb % 2],
        ).wait()

    return 0

  lax.fori_loop(0, PAIRS_PER_W, pair_body, 0)


@jax.jit
def kernel(x, tables):
  xi = x.astype(jnp.int32)
  tab_t = jnp.transpose(tables, (0, 2, 1))
  mesh = plsc.VectorSubcoreMesh(core_axis_name="c", subcore_axis_name="s")
  out_t = pl.kernel(
      _sc_body,
      out_type=jax.ShapeDtypeStruct((N_FIELDS, EMB, BATCH), jnp.float32),
      mesh=mesh,
      scratch_types=[
          pltpu.VMEM((VOCAB,), jnp.float32),
          pltpu.VMEM((2, BCHUNK), jnp.int32),
          pltpu.VMEM((2, BCHUNK), jnp.float32),
          pltpu.SemaphoreType.DMA((2,)),
          pltpu.SemaphoreType.DMA((2,)),
          pltpu.SemaphoreType.DMA((2,)),
      ],
      compiler_params=pltpu.CompilerParams(
          use_tc_tiling_on_sc=True, needs_layout_passes=False
      ),
  )(xi, tab_t)
  return jnp.transpose(out_t, (0, 2, 1))
